# Initial kernel scaffold; baseline (speedup 1.0000x reference)
import jax
import jax.numpy as jnp
from jax.experimental import pallas as pl


def kernel(text, emb_table, W1, b1, W2, b2):
    raise NotImplementedError("write your pallas kernel here")


# speedup vs baseline: 1.2650x; 1.2650x over previous
import functools

import jax
import jax.numpy as jnp
from jax import lax
from jax.experimental import pallas as pl
from jax.experimental.pallas import tpu as pltpu
from jax.experimental.pallas import tpu_sc as plsc

EMB = 32
HID = 128
OUT = 2
SEQ = 200
BATCH = 4096

NC = 2
NS = 16
NW = NC * NS
BPW = BATCH // NW
LANES = 16
VPR = EMB // LANES
VECS = BPW * VPR


def _pool_sums(text, emb_table):
    mesh = plsc.VectorSubcoreMesh(core_axis_name="c", subcore_axis_name="s")

    @functools.partial(
        pl.kernel,
        mesh=mesh,
        out_type=jax.ShapeDtypeStruct((BATCH, EMB), jnp.float32),
        scratch_types=[
            pltpu.VMEM((SEQ, BPW), jnp.int32),
            pltpu.VMEM((BPW, EMB), jnp.float32),
            pltpu.VMEM((BPW, EMB), jnp.float32),
            pltpu.SemaphoreType.DMA,
        ],
        compiler_params=pltpu.CompilerParams(use_tc_tiling_on_sc=False),
    )
    def pool(text_hbm, table_hbm, out_hbm, idx_v, acc_v, rows_v, sem):
        wid = lax.axis_index("s") * NC + lax.axis_index("c")
        base = wid * BPW
        pltpu.sync_copy(text_hbm.at[:, pl.ds(base, BPW)], idx_v)

        zero = jnp.zeros((LANES,), jnp.float32)

        def zbody(t, _):
            r = t // VPR
            c = (t % VPR) * LANES
            acc_v[r, pl.ds(c, LANES)] = zero
            return 0

        lax.fori_loop(0, VECS, zbody, 0)

        def sbody(s, _):
            pltpu.async_copy(table_hbm.at[idx_v.at[s]], rows_v, sem).wait()

            def abody(t, _):
                r = t // VPR
                c = (t % VPR) * LANES
                plsc.addupdate(acc_v.at[r, pl.ds(c, LANES)],
                               rows_v[r, pl.ds(c, LANES)])
                return 0

            lax.fori_loop(0, VECS, abody, 0)
            return 0

        lax.fori_loop(0, SEQ, sbody, 0)

        pltpu.sync_copy(acc_v, out_hbm.at[pl.ds(base, BPW)])

    return pool(text, emb_table)


def _mlp(sums, W1, b1, W2, b2):
    BN = 1024

    def mlp_body(x_ref, w1_ref, b1_ref, w2_ref, b2_ref, o_ref):
        x = x_ref[...]
        h = jnp.dot(x, w1_ref[...], preferred_element_type=jnp.float32)
        h = h * (1.0 / SEQ) + b1_ref[...]
        h = jnp.maximum(h, 0.0)
        o_ref[...] = (jnp.dot(h, w2_ref[...], preferred_element_type=jnp.float32)
                      + b2_ref[...])

    return pl.pallas_call(
        mlp_body,
        grid=(BATCH // BN,),
        in_specs=[
            pl.BlockSpec((BN, EMB), lambda i: (i, 0)),
            pl.BlockSpec((EMB, HID), lambda i: (0, 0)),
            pl.BlockSpec((1, HID), lambda i: (0, 0)),
            pl.BlockSpec((HID, OUT), lambda i: (0, 0)),
            pl.BlockSpec((1, OUT), lambda i: (0, 0)),
        ],
        out_specs=pl.BlockSpec((BN, OUT), lambda i: (i, 0)),
        out_shape=jax.ShapeDtypeStruct((BATCH, OUT), jnp.float32),
    )(sums, W1, b1.reshape(1, HID), W2, b2.reshape(1, OUT))


def kernel(text, emb_table, W1, b1, W2, b2):
    text = text.astype(jnp.int32)
    sums = _pool_sums(text, emb_table)
    return _mlp(sums, W1, b1, W2, b2)


# speedup vs baseline: 1.7677x; 1.3974x over previous
import functools

import jax
import jax.numpy as jnp
from jax import lax
from jax.experimental import pallas as pl
from jax.experimental.pallas import tpu as pltpu
from jax.experimental.pallas import tpu_sc as plsc

EMB = 32
HID = 128
OUT = 2
SEQ = 200
BATCH = 4096

NC = 2
NS = 16
NW = NC * NS
BPW = BATCH // NW
LANES = 16
VPR = EMB // LANES
VECS = BPW * VPR


def _pool_sums(text, emb_table):
    mesh = plsc.VectorSubcoreMesh(core_axis_name="c", subcore_axis_name="s")

    @functools.partial(
        pl.kernel,
        mesh=mesh,
        out_type=jax.ShapeDtypeStruct((BATCH, EMB), jnp.float32),
        scratch_types=[
            pltpu.VMEM((SEQ, BPW), jnp.int32),
            pltpu.VMEM((BPW, EMB), jnp.float32),
            pltpu.VMEM((2, BPW, EMB), jnp.float32),
            pltpu.SemaphoreType.DMA,
            pltpu.SemaphoreType.DMA,
        ],
        compiler_params=pltpu.CompilerParams(use_tc_tiling_on_sc=False),
    )
    def pool(text_hbm, table_hbm, out_hbm, idx_v, acc_v, rows_v, sem0, sem1):
        wid = lax.axis_index("s") * NC + lax.axis_index("c")
        base = wid * BPW
        pltpu.sync_copy(text_hbm.at[:, pl.ds(base, BPW)], idx_v)

        sems = (sem0, sem1)
        zero = jnp.zeros((LANES,), jnp.float32)

        def zbody(r, _):
            acc_v[r, pl.ds(0, LANES)] = zero
            acc_v[r, pl.ds(LANES, LANES)] = zero
            return 0

        lax.fori_loop(0, BPW, zbody, 0, unroll=8)

        def start(s, b):
            pltpu.make_async_copy(
                table_hbm.at[idx_v.at[s]], rows_v.at[b], sems[b]).start()

        def wait_acc(b):
            pltpu.make_async_copy(
                table_hbm.at[idx_v.at[0]], rows_v.at[b], sems[b]).wait()

            def abody(r, _):
                plsc.addupdate(acc_v.at[r, pl.ds(0, LANES)],
                               rows_v[b, r, pl.ds(0, LANES)])
                plsc.addupdate(acc_v.at[r, pl.ds(LANES, LANES)],
                               rows_v[b, r, pl.ds(LANES, LANES)])
                return 0

            lax.fori_loop(0, BPW, abody, 0, unroll=8)

        start(0, 0)
        start(1, 1)

        def gbody(g, _):
            s0 = 2 * g
            wait_acc(0)
            start(s0 + 2, 0)
            wait_acc(1)
            start(s0 + 3, 1)
            return 0

        lax.fori_loop(0, SEQ // 2 - 1, gbody, 0)
        wait_acc(0)
        wait_acc(1)

        pltpu.sync_copy(acc_v, out_hbm.at[pl.ds(base, BPW)])

    return pool(text, emb_table)


def _mlp(sums, W1, b1, W2, b2):
    BN = 1024

    def mlp_body(x_ref, w1_ref, b1_ref, w2_ref, b2_ref, o_ref):
        x = x_ref[...]
        h = jnp.dot(x, w1_ref[...], preferred_element_type=jnp.float32)
        h = h * (1.0 / SEQ) + b1_ref[...]
        h = jnp.maximum(h, 0.0)
        o_ref[...] = (jnp.dot(h, w2_ref[...], preferred_element_type=jnp.float32)
                      + b2_ref[...])

    return pl.pallas_call(
        mlp_body,
        grid=(BATCH // BN,),
        in_specs=[
            pl.BlockSpec((BN, EMB), lambda i: (i, 0)),
            pl.BlockSpec((EMB, HID), lambda i: (0, 0)),
            pl.BlockSpec((1, HID), lambda i: (0, 0)),
            pl.BlockSpec((HID, OUT), lambda i: (0, 0)),
            pl.BlockSpec((1, OUT), lambda i: (0, 0)),
        ],
        out_specs=pl.BlockSpec((BN, OUT), lambda i: (i, 0)),
        out_shape=jax.ShapeDtypeStruct((BATCH, OUT), jnp.float32),
    )(sums, W1, b1.reshape(1, HID), W2, b2.reshape(1, OUT))


def kernel(text, emb_table, W1, b1, W2, b2):
    text = text.astype(jnp.int32)
    sums = _pool_sums(text, emb_table)
    return _mlp(sums, W1, b1, W2, b2)
